# Initial kernel scaffold; baseline (speedup 1.0000x reference)
import jax
import jax.numpy as jnp
from jax.experimental import pallas as pl


def kernel(node_features, timestamps, params, edge_index):
    raise NotImplementedError("write your pallas kernel here")


# speedup vs baseline: 1.0094x; 1.0094x over previous
import functools

import jax
import jax.numpy as jnp
import numpy as np
from jax.experimental import pallas as pl


def _ln(x, g, b):
    m = x.mean(-1, keepdims=True)
    v = ((x - m) ** 2).mean(-1, keepdims=True)
    return (x - m) / jnp.sqrt(v + 1e-5) * g + b


def _te_body(ts_ref, freq_ref, wte0_ref, wte1_ref, te_ref, teh0_ref, teh1_ref):
    te = jnp.cos(ts_ref[...] * freq_ref[...])
    te_ref[...] = te
    teh0_ref[...] = te @ wte0_ref[...]
    teh1_ref[...] = te @ wte1_ref[...]


def _te_pipeline(ts2, freq2, wte0, wte1, BE=2000):
    E = ts2.shape[0]
    TD = freq2.shape[1]
    HD = wte0.shape[1]
    G = E // BE
    return pl.pallas_call(
        _te_body,
        grid=(G,),
        in_specs=[
            pl.BlockSpec((BE, 1), lambda i: (i, 0)),
            pl.BlockSpec((1, TD), lambda i: (0, 0)),
            pl.BlockSpec((TD, HD), lambda i: (0, 0)),
            pl.BlockSpec((TD, HD), lambda i: (0, 0)),
        ],
        out_specs=[
            pl.BlockSpec((BE, TD), lambda i: (i, 0)),
            pl.BlockSpec((BE, HD), lambda i: (i, 0)),
            pl.BlockSpec((BE, HD), lambda i: (i, 0)),
        ],
        out_shape=[
            jax.ShapeDtypeStruct((E, TD), jnp.float32),
            jax.ShapeDtypeStruct((E, HD), jnp.float32),
            jax.ShapeDtypeStruct((E, HD), jnp.float32),
        ],
    )(ts2, freq2, wte0, wte1)


def _x0_body(nf_ref, w_ref, b_ref, o_ref):
    o_ref[...] = nf_ref[...] @ w_ref[...] + b_ref[...]


def _x0(nf, w, b2, BN=400):
    N, D = nf.shape
    HD = w.shape[1]
    return pl.pallas_call(
        _x0_body,
        grid=(N // BN,),
        in_specs=[
            pl.BlockSpec((BN, D), lambda i: (i, 0)),
            pl.BlockSpec((D, HD), lambda i: (0, 0)),
            pl.BlockSpec((1, HD), lambda i: (0, 0)),
        ],
        out_specs=pl.BlockSpec((BN, HD), lambda i: (i, 0)),
        out_shape=jax.ShapeDtypeStruct((N, HD), jnp.float32),
    )(nf, w, b2)


def _qkv_body(x_ref, wq_ref, wk_ref, wv_ref, q_ref, k_ref, v_ref):
    x = x_ref[...]
    q_ref[...] = x @ wq_ref[...]
    k_ref[...] = x @ wk_ref[...]
    v_ref[...] = x @ wv_ref[...]


def _qkv(x, wq, wk, wv, BN=400):
    N, HD = x.shape
    mk = lambda: jax.ShapeDtypeStruct((N, HD), jnp.float32)
    wspec = pl.BlockSpec((HD, HD), lambda i: (0, 0))
    xspec = pl.BlockSpec((BN, HD), lambda i: (i, 0))
    return pl.pallas_call(
        _qkv_body,
        grid=(N // BN,),
        in_specs=[xspec, wspec, wspec, wspec],
        out_specs=[xspec, xspec, xspec],
        out_shape=[mk(), mk(), mk()],
    )(x, wq, wk, wv)


def _post_body(x_ref, agg_ref, wo_ref, ln1g_ref, ln1b_ref, w1_ref, b1_ref,
               w2_ref, b2_ref, ln2g_ref, ln2b_ref, o_ref):
    x = x_ref[...]
    x = _ln(x + agg_ref[...] @ wo_ref[...], ln1g_ref[...], ln1b_ref[...])
    ff = jnp.maximum(x @ w1_ref[...] + b1_ref[...], 0.0) @ w2_ref[...] + b2_ref[...]
    o_ref[...] = _ln(x + ff, ln2g_ref[...], ln2b_ref[...])


def _post(x, agg, wo, ln1g, ln1b, w1, b1, w2, b2, ln2g, ln2b, BN=400):
    N, HD = x.shape
    F = w1.shape[1]
    xspec = pl.BlockSpec((BN, HD), lambda i: (i, 0))
    return pl.pallas_call(
        _post_body,
        grid=(N // BN,),
        in_specs=[
            xspec, xspec,
            pl.BlockSpec((HD, HD), lambda i: (0, 0)),
            pl.BlockSpec((1, HD), lambda i: (0, 0)),
            pl.BlockSpec((1, HD), lambda i: (0, 0)),
            pl.BlockSpec((HD, F), lambda i: (0, 0)),
            pl.BlockSpec((1, F), lambda i: (0, 0)),
            pl.BlockSpec((F, HD), lambda i: (0, 0)),
            pl.BlockSpec((1, HD), lambda i: (0, 0)),
            pl.BlockSpec((1, HD), lambda i: (0, 0)),
            pl.BlockSpec((1, HD), lambda i: (0, 0)),
        ],
        out_specs=xspec,
        out_shape=jax.ShapeDtypeStruct((N, HD), jnp.float32),
    )(x, agg, wo, ln1g, ln1b, w1, b1, w2, b2, ln2g, ln2b)


def _latent_body(x_ref, wmu_ref, bmu_ref, wlv_ref, blv_ref, eps_ref,
                 mean_ref, lv_ref, z_ref, kl_ref):
    x = x_ref[...]
    mean = x @ wmu_ref[...] + bmu_ref[...]
    lv = x @ wlv_ref[...] + blv_ref[...]
    mean_ref[...] = mean
    lv_ref[...] = lv
    lvc = jnp.clip(lv, -10.0, 10.0)
    z_ref[...] = mean + jnp.exp(0.5 * lvc) * eps_ref[...]
    blk = jnp.sum(1.0 + lv - mean * mean - jnp.exp(lvc))

    @pl.when(pl.program_id(0) == 0)
    def _():
        kl_ref[...] = jnp.zeros_like(kl_ref)

    kl_ref[...] += blk.reshape(1, 1)


def _latent(x, wmu, bmu, wlv, blv, eps, BN=400):
    N, HD = x.shape
    xspec = pl.BlockSpec((BN, HD), lambda i: (i, 0))
    wspec = pl.BlockSpec((HD, HD), lambda i: (0, 0))
    bspec = pl.BlockSpec((1, HD), lambda i: (0, 0))
    mk = lambda: jax.ShapeDtypeStruct((N, HD), jnp.float32)
    return pl.pallas_call(
        _latent_body,
        grid=(N // BN,),
        in_specs=[xspec, wspec, bspec, wspec, bspec, xspec],
        out_specs=[xspec, xspec, xspec,
                   pl.BlockSpec((1, 1), lambda i: (0, 0))],
        out_shape=[mk(), mk(), mk(),
                   jax.ShapeDtypeStruct((1, 1), jnp.float32)],
    )(x, wmu, bmu, wlv, blv, eps)


def _diff_body(z_ref, nb_ref, deg_ref, w_ref, b_ref, o_ref):
    nb = nb_ref[...] / deg_ref[...]
    o_ref[...] = z_ref[...] + jnp.tanh(nb @ w_ref[...] + b_ref[...])


def _diff_step(z, nbsum, deg2, w, b2, BN=400):
    N, HD = z.shape
    xspec = pl.BlockSpec((BN, HD), lambda i: (i, 0))
    return pl.pallas_call(
        _diff_body,
        grid=(N // BN,),
        in_specs=[
            xspec, xspec,
            pl.BlockSpec((BN, 1), lambda i: (i, 0)),
            pl.BlockSpec((HD, HD), lambda i: (0, 0)),
            pl.BlockSpec((1, HD), lambda i: (0, 0)),
        ],
        out_specs=xspec,
        out_shape=jax.ShapeDtypeStruct((N, HD), jnp.float32),
    )(z, nbsum, deg2, w, b2)


def _final_body(z_ref, g_ref, b_ref, w_ref, ob_ref, o_ref):
    o_ref[...] = _ln(z_ref[...], g_ref[...], b_ref[...]) @ w_ref[...] + ob_ref[...]


def _final(z, g, b2, w, ob, BN=400):
    N, HD = z.shape
    xspec = pl.BlockSpec((BN, HD), lambda i: (i, 0))
    return pl.pallas_call(
        _final_body,
        grid=(N // BN,),
        in_specs=[
            xspec,
            pl.BlockSpec((1, HD), lambda i: (0, 0)),
            pl.BlockSpec((1, HD), lambda i: (0, 0)),
            pl.BlockSpec((HD, HD), lambda i: (0, 0)),
            pl.BlockSpec((1, HD), lambda i: (0, 0)),
        ],
        out_specs=xspec,
        out_shape=jax.ShapeDtypeStruct((N, HD), jnp.float32),
    )(z, g, b2, w, ob)


def kernel(node_features, timestamps, params, edge_index):
    N, D = node_features.shape
    E = timestamps.shape[0]
    TD = params['time_freq'].shape[0]
    L, HD, _ = params['wq'].shape
    NH = 4
    DH = HD // NH
    S = params['w_diff'].shape[0]

    src = edge_index[0]
    dst = edge_index[1]
    r2 = lambda a: a.reshape(1, -1)

    te, teh0, teh1 = _te_pipeline(
        timestamps.reshape(E, 1), r2(params['time_freq']),
        params['wte'][0], params['wte'][1])
    tehs = (teh0, teh1)

    x = _x0(node_features, params['node_w'], r2(params['node_b']))

    for l in range(L):
        q, k, v = _qkv(x, params['wq'][l], params['wk'][l], params['wv'][l])
        te_h = tehs[l]
        qe = (q[dst]).reshape(E, NH, DH)
        ke = (k[src] + te_h).reshape(E, NH, DH)
        ve = (v[src] + te_h).reshape(E, NH, DH)
        scores = (qe * ke).sum(-1) / np.sqrt(DH)
        smax = jax.ops.segment_max(scores, dst, num_segments=N)
        smax = jnp.where(jnp.isfinite(smax), smax, 0.0)
        ex = jnp.exp(scores - smax[dst])
        denom = jax.ops.segment_sum(ex, dst, num_segments=N) + 1e-9
        alpha = ex / denom[dst]
        agg = jax.ops.segment_sum(alpha[:, :, None] * ve, dst,
                                  num_segments=N).reshape(N, HD)
        x = _post(x, agg, params['wo'][l], r2(params['ln1_g'][l]),
                  r2(params['ln1_b'][l]), params['ffn_w1'][l],
                  r2(params['ffn_b1'][l]), params['ffn_w2'][l],
                  r2(params['ffn_b2'][l]), r2(params['ln2_g'][l]),
                  r2(params['ln2_b'][l]))

    eps = jax.random.normal(jax.random.key(42), (N, HD), dtype=jnp.float32)
    mean, logvar, z, kl_sum = _latent(
        x, params['w_mu'], r2(params['b_mu']),
        params['w_lv'], r2(params['b_lv']), eps)
    kl = -0.5 * kl_sum[0, 0] / (N * HD)

    deg = jax.ops.segment_sum(jnp.ones((E,), jnp.float32), dst,
                              num_segments=N) + 1.0
    for s in range(S):
        nbsum = jax.ops.segment_sum(z[src], dst, num_segments=N)
        z = _diff_step(z, nbsum, deg.reshape(N, 1),
                       params['w_diff'][s], r2(params['b_diff'][s]))

    emb = _final(z, r2(params['fin_g']), r2(params['fin_b']),
                 params['out_w'], r2(params['out_b']))
    return emb, mean, logvar, kl, te


# speedup vs baseline: 8.2117x; 8.1351x over previous
import functools

import jax
import jax.numpy as jnp
import numpy as np
from jax import lax
from jax.experimental import pallas as pl
from jax.experimental.pallas import tpu as pltpu
from jax.experimental.pallas import tpu_sc as plsc


_SC_B = 128


def _zero_vmem_rows(ref, nrows, width):
    z16 = jnp.zeros((16,), jnp.float32)

    def body(i, carry):
        for d in range(width // 16):
            ref[i, pl.ds(16 * d, 16)] = z16
        return carry

    lax.fori_loop(0, nrows, body, 0)


def _stripe_layout(N):
    main = (N // (16 * 8)) * 8
    extra = N - 16 * main
    return main, extra


def _zero_stripe(s, rows, acc, N):
    main, extra = _stripe_layout(N)
    B = _SC_B
    off = s * main
    nch = (main + B - 1) // B
    for j in range(nch):
        r0 = j * B
        rlen = min(B, main - r0)
        pltpu.sync_copy(rows.at[pl.ds(0, rlen)], acc.at[pl.ds(off + r0, rlen)])
    if extra:
        @pl.when(s == 15)
        def _():
            pltpu.sync_copy(rows.at[pl.ds(0, extra)],
                            acc.at[pl.ds(16 * main, extra)])


def _dump_stripe(c, s, acc, out_hbm, N):
    main, extra = _stripe_layout(N)
    off = s * main
    pltpu.sync_copy(acc.at[pl.ds(off, main)],
                    out_hbm.at[c].at[pl.ds(off, main)])
    if extra:
        @pl.when(s == 15)
        def _():
            pltpu.sync_copy(acc.at[pl.ds(16 * main, extra)],
                            out_hbm.at[c].at[pl.ds(16 * main, extra)])


def _seg_rows_kernel(nblocks_total, N, W, table_hbm, src_hbm, dst_hbm,
                     out_hbm, sidx, didx, rows, acc, sem):
    c = lax.axis_index("c")
    s = lax.axis_index("s")
    wid = s * 2 + c
    B = _SC_B

    _zero_vmem_rows(rows, B, W)
    _zero_stripe(s, rows, acc, N)
    plsc.subcore_barrier()

    def do_block(base):
        pltpu.sync_copy(src_hbm.at[pl.ds(base, B)], sidx)
        pltpu.sync_copy(dst_hbm.at[pl.ds(base, B)], didx)
        pltpu.async_copy(table_hbm.at[sidx], rows, sem).wait()
        pltpu.sync_copy(rows, acc.at[didx], add=True)

    nblk = nblocks_total // 32

    def body(j, carry):
        do_block((wid + 32 * j) * B)
        return carry

    lax.fori_loop(0, nblk, body, 0)
    for t, blk in enumerate(range(nblk * 32, nblocks_total)):
        @pl.when(wid == t)
        def _(blk=blk):
            do_block(blk * B)
    plsc.subcore_barrier()
    _dump_stripe(c, s, acc, out_hbm, N)


def _flat_layout(M):
    main = (M // (16 * 128)) * 128
    extra = M - 16 * main
    return main, extra


def _zero_flat_stripe(s, zsrc, acc_flat, M):
    B = _SC_B
    main, extra = _flat_layout(M)
    off = s * main
    for j in range((main + B - 1) // B):
        r0 = j * B
        rlen = min(B, main - r0)
        pltpu.sync_copy(zsrc.at[pl.ds(0, rlen)], acc_flat.at[pl.ds(off + r0, rlen)])
    if extra:
        @pl.when(s == 15)
        def _():
            for j in range((extra + B - 1) // B):
                r0 = j * B
                rlen = min(B, extra - r0)
                pltpu.sync_copy(zsrc.at[pl.ds(0, rlen)],
                                acc_flat.at[pl.ds(16 * main + r0, rlen)])


def _dump_flat_stripe(c, s, acc_flat, out_hbm, M):
    main, extra = _flat_layout(M)
    off = s * main
    pltpu.sync_copy(acc_flat.at[pl.ds(off, main)],
                    out_hbm.at[c].at[pl.ds(off, main)])
    if extra:
        @pl.when(s == 15)
        def _():
            pltpu.sync_copy(acc_flat.at[pl.ds(16 * main, extra)],
                            out_hbm.at[c].at[pl.ds(16 * main, extra)])


def _attn_prod_body(nblocks_total, q_hbm, k_hbm, teh_hbm, src_hbm, dst_hbm,
                    prod_out, sidx, didx, qrows, krows, terows, prows, sem):
    c = lax.axis_index("c")
    s = lax.axis_index("s")
    wid = s * 2 + c
    B = _SC_B

    def do_block(bidx):
        base = bidx * B
        pltpu.sync_copy(src_hbm.at[pl.ds(base, B)], sidx)
        pltpu.sync_copy(dst_hbm.at[pl.ds(base, B)], didx)
        pltpu.async_copy(q_hbm.at[didx], qrows, sem).wait()
        pltpu.async_copy(k_hbm.at[sidx], krows, sem).wait()
        pltpu.sync_copy(teh_hbm.at[pl.ds(base, B)], terows)

        def ebody(e, carry):
            for d in range(8):
                qv = qrows[e, pl.ds(16 * d, 16)]
                kv = krows[e, pl.ds(16 * d, 16)]
                tv = terows[e, pl.ds(16 * d, 16)]
                prows[e, pl.ds(16 * d, 16)] = qv * (kv + tv)
            return carry

        lax.fori_loop(0, B, ebody, 0)
        pltpu.sync_copy(prows, prod_out.at[pl.ds(base, B)])

    nblk = nblocks_total // 32

    def body(j, carry):
        do_block(wid + 32 * j)
        return carry

    lax.fori_loop(0, nblk, body, 0)
    for t, blk in enumerate(range(nblk * 32, nblocks_total)):
        @pl.when(wid == t)
        def _(blk=blk):
            do_block(jnp.int32(blk))


def _attn_prod(q, k, teh, src, dst):
    N, HD = q.shape
    E = src.shape[0]
    nblocks = E // _SC_B
    mesh = plsc.VectorSubcoreMesh(core_axis_name="c", subcore_axis_name="s")
    f = functools.partial(
        pl.kernel,
        out_type=jax.ShapeDtypeStruct((E, HD), jnp.float32),
        mesh=mesh,
        scratch_types=[
            pltpu.VMEM((_SC_B,), jnp.int32),
            pltpu.VMEM((_SC_B,), jnp.int32),
            pltpu.VMEM((_SC_B, HD), jnp.float32),
            pltpu.VMEM((_SC_B, HD), jnp.float32),
            pltpu.VMEM((_SC_B, HD), jnp.float32),
            pltpu.VMEM((_SC_B, HD), jnp.float32),
            pltpu.SemaphoreType.DMA,
        ],
    )(functools.partial(_attn_prod_body, nblocks))
    return f(q, k, teh, src, dst)


def _scores_tc_body(prod_ref, sel_ref, st_ref, gm_ref):
    s = (prod_ref[...] @ sel_ref[...]) * (1.0 / np.sqrt(32.0))
    st_ref[...] = s.T
    m4 = jnp.max(s, axis=0, keepdims=True)
    mw = jnp.broadcast_to(m4.T, (4, 16))

    @pl.when(pl.program_id(0) == 0)
    def _():
        gm_ref[...] = jnp.full((4, 16), -3.0e38, jnp.float32)

    gm_ref[...] = jnp.maximum(gm_ref[...], mw)


def _scores_tc(prod, sel4, BE=3200):
    E, HD = prod.shape
    return pl.pallas_call(
        _scores_tc_body,
        grid=(E // BE,),
        in_specs=[
            pl.BlockSpec((BE, HD), lambda i: (i, 0)),
            pl.BlockSpec((HD, 4), lambda i: (0, 0)),
        ],
        out_specs=[
            pl.BlockSpec((4, BE), lambda i: (0, i)),
            pl.BlockSpec((4, 16), lambda i: (0, 0)),
        ],
        out_shape=[
            jax.ShapeDtypeStruct((4, E), jnp.float32),
            jax.ShapeDtypeStruct((4, 16), jnp.float32),
        ],
    )(prod, sel4)


def _attn_agg_body(nblocks_total, N, v_hbm, teh_hbm, src_hbm, dst_hbm,
                   s_hbm, gmax_hbm, agg_out, sidx, didx, vrows,
                   terows, srows, sblk, exb3, gmv, agg_acc, sem):
    c = lax.axis_index("c")
    s = lax.axis_index("s")
    wid = s * 2 + c
    B = _SC_B

    _zero_vmem_rows(srows, B, 128)
    _zero_stripe(s, srows, agg_acc, N)
    plsc.subcore_barrier()

    pltpu.sync_copy(gmax_hbm, gmv)
    gmb = [gmv[h, pl.ds(0, 16)] for h in range(4)]

    def do_block(bidx):
        base = bidx * B
        pltpu.sync_copy(src_hbm.at[pl.ds(base, B)], sidx)
        pltpu.sync_copy(dst_hbm.at[pl.ds(base, B)], didx)
        pltpu.async_copy(v_hbm.at[sidx], vrows, sem).wait()
        pltpu.sync_copy(teh_hbm.at[pl.ds(base, B)], terows)
        pltpu.sync_copy(s_hbm.at[:, pl.ds(base, B)], sblk)
        for g in range(8):
            for h in range(4):
                ex = jnp.exp(sblk[h, pl.ds(g * 16, 16)] - gmb[h])
                exb3[h, g, pl.ds(0, 16)] = ex

        def gbody2(g, carry):
            exvs = [exb3[h, g, pl.ds(0, 16)] for h in range(4)]
            for i in range(16):
                e = g * 16 + i
                for h in range(4):
                    exs = exvs[h][i]
                    for d in (2 * h, 2 * h + 1):
                        vv = vrows[e, pl.ds(16 * d, 16)]
                        tv = terows[e, pl.ds(16 * d, 16)]
                        srows[e, pl.ds(16 * d, 16)] = (vv + tv) * exs
            return carry

        lax.fori_loop(0, 8, gbody2, 0)
        pltpu.sync_copy(srows, agg_acc.at[didx], add=True)

    nblk = nblocks_total // 32

    def body(j, carry):
        do_block(wid + 32 * j)
        return carry

    lax.fori_loop(0, nblk, body, 0)
    for t, blk in enumerate(range(nblk * 32, nblocks_total)):
        @pl.when(wid == t)
        def _(blk=blk):
            do_block(jnp.int32(blk))
    plsc.subcore_barrier()
    _dump_stripe(c, s, acc=agg_acc, out_hbm=agg_out, N=N)


def _attn_agg(v, teh, src, dst, s_hbm, gmax16, N):
    E = src.shape[0]
    HD = v.shape[1]
    nblocks = E // _SC_B
    mesh = plsc.VectorSubcoreMesh(core_axis_name="c", subcore_axis_name="s")
    f = functools.partial(
        pl.kernel,
        out_type=jax.ShapeDtypeStruct((2, N, HD), jnp.float32),
        mesh=mesh,
        scratch_types=[
            pltpu.VMEM((_SC_B,), jnp.int32),
            pltpu.VMEM((_SC_B,), jnp.int32),
            pltpu.VMEM((_SC_B, HD), jnp.float32),
            pltpu.VMEM((_SC_B, HD), jnp.float32),
            pltpu.VMEM((_SC_B, HD), jnp.float32),
            pltpu.VMEM((4, _SC_B), jnp.float32),
            pltpu.VMEM((4, 8, 16), jnp.float32),
            pltpu.VMEM((4, 16), jnp.float32),
            pltpu.VMEM_SHARED((N, HD), jnp.float32),
            pltpu.SemaphoreType.DMA,
        ],
    )(functools.partial(_attn_agg_body, nblocks, N))
    return f(v, teh, src, dst, s_hbm, gmax16)


def _segment_sum_rows(table, src, dst, N):
    E = src.shape[0]
    W = table.shape[1]
    assert E % _SC_B == 0
    nblocks = E // _SC_B
    mesh = plsc.VectorSubcoreMesh(core_axis_name="c", subcore_axis_name="s")
    f = functools.partial(
        pl.kernel,
        out_type=jax.ShapeDtypeStruct((2, N, W), jnp.float32),
        mesh=mesh,
        scratch_types=[
            pltpu.VMEM((_SC_B,), jnp.int32),
            pltpu.VMEM((_SC_B,), jnp.int32),
            pltpu.VMEM((_SC_B, W), jnp.float32),
            pltpu.VMEM_SHARED((N, W), jnp.float32),
            pltpu.SemaphoreType.DMA,
        ],
    )(functools.partial(_seg_rows_kernel, nblocks, N, W))
    return f(table, src, dst)


def _ln(x, g, b):
    m = x.mean(-1, keepdims=True)
    v = ((x - m) ** 2).mean(-1, keepdims=True)
    return (x - m) / jnp.sqrt(v + 1e-5) * g + b


def _te_body(ts_ref, freq_ref, wte0_ref, wte1_ref, te_ref, teh0_ref, teh1_ref):
    te = jnp.cos(ts_ref[...] * freq_ref[...])
    te_ref[...] = te
    teh0_ref[...] = te @ wte0_ref[...]
    teh1_ref[...] = te @ wte1_ref[...]


def _te_pipeline(ts2, freq2, wte0, wte1, BE=2000):
    E = ts2.shape[0]
    TD = freq2.shape[1]
    HD = wte0.shape[1]
    G = E // BE
    return pl.pallas_call(
        _te_body,
        grid=(G,),
        in_specs=[
            pl.BlockSpec((BE, 1), lambda i: (i, 0)),
            pl.BlockSpec((1, TD), lambda i: (0, 0)),
            pl.BlockSpec((TD, HD), lambda i: (0, 0)),
            pl.BlockSpec((TD, HD), lambda i: (0, 0)),
        ],
        out_specs=[
            pl.BlockSpec((BE, TD), lambda i: (i, 0)),
            pl.BlockSpec((BE, HD), lambda i: (i, 0)),
            pl.BlockSpec((BE, HD), lambda i: (i, 0)),
        ],
        out_shape=[
            jax.ShapeDtypeStruct((E, TD), jnp.float32),
            jax.ShapeDtypeStruct((E, HD), jnp.float32),
            jax.ShapeDtypeStruct((E, HD), jnp.float32),
        ],
    )(ts2, freq2, wte0, wte1)


def _x0_body(nf_ref, w_ref, b_ref, o_ref):
    o_ref[...] = nf_ref[...] @ w_ref[...] + b_ref[...]


def _x0(nf, w, b2, BN=400):
    N, D = nf.shape
    HD = w.shape[1]
    return pl.pallas_call(
        _x0_body,
        grid=(N // BN,),
        in_specs=[
            pl.BlockSpec((BN, D), lambda i: (i, 0)),
            pl.BlockSpec((D, HD), lambda i: (0, 0)),
            pl.BlockSpec((1, HD), lambda i: (0, 0)),
        ],
        out_specs=pl.BlockSpec((BN, HD), lambda i: (i, 0)),
        out_shape=jax.ShapeDtypeStruct((N, HD), jnp.float32),
    )(nf, w, b2)


def _qkv_body(x_ref, wq_ref, wk_ref, wv_ref, q_ref, k_ref, v_ref):
    x = x_ref[...]
    q_ref[...] = x @ wq_ref[...]
    k_ref[...] = x @ wk_ref[...]
    v_ref[...] = x @ wv_ref[...]


def _qkv(x, wq, wk, wv, BN=400):
    N, HD = x.shape
    mk = lambda: jax.ShapeDtypeStruct((N, HD), jnp.float32)
    wspec = pl.BlockSpec((HD, HD), lambda i: (0, 0))
    xspec = pl.BlockSpec((BN, HD), lambda i: (i, 0))
    return pl.pallas_call(
        _qkv_body,
        grid=(N // BN,),
        in_specs=[xspec, wspec, wspec, wspec],
        out_specs=[xspec, xspec, xspec],
        out_shape=[mk(), mk(), mk()],
    )(x, wq, wk, wv)


def _post_body(x_ref, aggp_ref, denp_ref, exp_ref, wo_ref, ln1g_ref,
               ln1b_ref, w1_ref, b1_ref, w2_ref, b2_ref, ln2g_ref,
               ln2b_ref, o_ref):
    x = x_ref[...]
    den = denp_ref[0, :, :4] + denp_ref[1, :, :4]
    den_wide = den @ exp_ref[...]
    den_wide = jnp.where(den_wide == 0.0, 1.0, den_wide)
    agg = (aggp_ref[0] + aggp_ref[1]) / den_wide
    x = _ln(x + agg @ wo_ref[...], ln1g_ref[...], ln1b_ref[...])
    ff = jnp.maximum(x @ w1_ref[...] + b1_ref[...], 0.0) @ w2_ref[...] + b2_ref[...]
    o_ref[...] = _ln(x + ff, ln2g_ref[...], ln2b_ref[...])


def _post(x, aggp, denp, expander, wo, ln1g, ln1b, w1, b1, w2, b2, ln2g,
          ln2b, BN=400):
    N, HD = x.shape
    F = w1.shape[1]
    xspec = pl.BlockSpec((BN, HD), lambda i: (i, 0))
    return pl.pallas_call(
        _post_body,
        grid=(N // BN,),
        in_specs=[
            xspec,
            pl.BlockSpec((2, BN, HD), lambda i: (0, i, 0)),
            pl.BlockSpec((2, BN, 8), lambda i: (0, i, 0)),
            pl.BlockSpec((4, HD), lambda i: (0, 0)),
            pl.BlockSpec((HD, HD), lambda i: (0, 0)),
            pl.BlockSpec((1, HD), lambda i: (0, 0)),
            pl.BlockSpec((1, HD), lambda i: (0, 0)),
            pl.BlockSpec((HD, F), lambda i: (0, 0)),
            pl.BlockSpec((1, F), lambda i: (0, 0)),
            pl.BlockSpec((F, HD), lambda i: (0, 0)),
            pl.BlockSpec((1, HD), lambda i: (0, 0)),
            pl.BlockSpec((1, HD), lambda i: (0, 0)),
            pl.BlockSpec((1, HD), lambda i: (0, 0)),
        ],
        out_specs=xspec,
        out_shape=jax.ShapeDtypeStruct((N, HD), jnp.float32),
    )(x, aggp, denp, expander, wo, ln1g, ln1b, w1, b1, w2, b2, ln2g, ln2b)


def _latent_body(x_ref, wmu_ref, bmu_ref, wlv_ref, blv_ref, eps_ref,
                 mean_ref, lv_ref, z_ref, kl_ref):
    x = x_ref[...]
    mean = x @ wmu_ref[...] + bmu_ref[...]
    lv = x @ wlv_ref[...] + blv_ref[...]
    mean_ref[...] = mean
    lv_ref[...] = lv
    lvc = jnp.clip(lv, -10.0, 10.0)
    z_ref[...] = mean + jnp.exp(0.5 * lvc) * eps_ref[...]
    blk = jnp.sum(1.0 + lv - mean * mean - jnp.exp(lvc))

    @pl.when(pl.program_id(0) == 0)
    def _():
        kl_ref[...] = jnp.zeros_like(kl_ref)

    kl_ref[...] += blk.reshape(1, 1)


def _latent(x, wmu, bmu, wlv, blv, eps, BN=400):
    N, HD = x.shape
    xspec = pl.BlockSpec((BN, HD), lambda i: (i, 0))
    wspec = pl.BlockSpec((HD, HD), lambda i: (0, 0))
    bspec = pl.BlockSpec((1, HD), lambda i: (0, 0))
    mk = lambda: jax.ShapeDtypeStruct((N, HD), jnp.float32)
    return pl.pallas_call(
        _latent_body,
        grid=(N // BN,),
        in_specs=[xspec, wspec, bspec, wspec, bspec, xspec],
        out_specs=[xspec, xspec, xspec,
                   pl.BlockSpec((1, 1), lambda i: (0, 0))],
        out_shape=[mk(), mk(), mk(),
                   jax.ShapeDtypeStruct((1, 1), jnp.float32)],
    )(x, wmu, bmu, wlv, blv, eps)


def _diff_body(z_ref, nb0_ref, nb1_ref, degp_ref, w_ref, b_ref, o_ref):
    dp = degp_ref[...]
    deg = dp[0, :, 4:5] + dp[1, :, 4:5] + 1.0
    nb = (nb0_ref[0] + nb1_ref[0]) / deg
    o_ref[...] = z_ref[...] + jnp.tanh(nb @ w_ref[...] + b_ref[...])


def _diff_step(z, nbp, degp, w, b2, BN=400):
    N, HD = z.shape
    W = degp.shape[2]
    xspec = pl.BlockSpec((BN, HD), lambda i: (i, 0))
    return pl.pallas_call(
        _diff_body,
        grid=(N // BN,),
        in_specs=[
            xspec,
            pl.BlockSpec((1, BN, HD), lambda i: (0, i, 0)),
            pl.BlockSpec((1, BN, HD), lambda i: (1, i, 0)),
            pl.BlockSpec((2, BN, W), lambda i: (0, i, 0)),
            pl.BlockSpec((HD, HD), lambda i: (0, 0)),
            pl.BlockSpec((1, HD), lambda i: (0, 0)),
        ],
        out_specs=xspec,
        out_shape=jax.ShapeDtypeStruct((N, HD), jnp.float32),
    )(z, nbp, nbp, degp, w, b2)


def _final_body(z_ref, g_ref, b_ref, w_ref, ob_ref, o_ref):
    o_ref[...] = _ln(z_ref[...], g_ref[...], b_ref[...]) @ w_ref[...] + ob_ref[...]


def _final(z, g, b2, w, ob, BN=400):
    N, HD = z.shape
    xspec = pl.BlockSpec((BN, HD), lambda i: (i, 0))
    return pl.pallas_call(
        _final_body,
        grid=(N // BN,),
        in_specs=[
            xspec,
            pl.BlockSpec((1, HD), lambda i: (0, 0)),
            pl.BlockSpec((1, HD), lambda i: (0, 0)),
            pl.BlockSpec((HD, HD), lambda i: (0, 0)),
            pl.BlockSpec((1, HD), lambda i: (0, 0)),
        ],
        out_specs=xspec,
        out_shape=jax.ShapeDtypeStruct((N, HD), jnp.float32),
    )(z, g, b2, w, ob)


def kernel(node_features, timestamps, params, edge_index):
    N, D = node_features.shape
    E = timestamps.shape[0]
    TD = params['time_freq'].shape[0]
    L, HD, _ = params['wq'].shape
    NH = 4
    DH = HD // NH
    S = params['w_diff'].shape[0]

    src = edge_index[0]
    dst = edge_index[1]
    r2 = lambda a: a.reshape(1, -1)

    te, teh0, teh1 = _te_pipeline(
        timestamps.reshape(E, 1), r2(params['time_freq']),
        params['wte'][0], params['wte'][1])
    tehs = (teh0, teh1)

    x = _x0(node_features, params['node_w'], r2(params['node_b']))

    expander = jnp.asarray(np.kron(np.eye(4), np.ones((1, 32))),
                           dtype=jnp.float32)
    denp0 = None
    for l in range(L):
        q, k, v = _qkv(x, params['wq'][l], params['wk'][l], params['wv'][l])
        te_h = tehs[l]
        prod = _attn_prod(q, k, te_h, src, dst)
        s_hbm, gmax16 = _scores_tc(prod, expander.T)
        ex4 = jnp.exp(s_hbm - gmax16[:, :1]).T
        aggr = jax.ops.segment_sum((ex4 @ expander) * (v[src] + te_h), dst,
                                   num_segments=N)
        aggp = jnp.stack([aggr, jnp.zeros_like(aggr)])
        denom = jax.ops.segment_sum(ex4, dst, num_segments=N)
        if denp0 is None:
            degc = jax.ops.segment_sum(jnp.ones((E,), jnp.float32), dst,
                                       num_segments=N)
        denp = jnp.zeros((2, N, 8), jnp.float32)
        denp = denp.at[0, :, :4].set(denom).at[0, :, 4].set(degc)
        if denp0 is None:
            denp0 = denp
        x = _post(x, aggp, denp, expander, params['wo'][l],
                  r2(params['ln1_g'][l]),
                  r2(params['ln1_b'][l]), params['ffn_w1'][l],
                  r2(params['ffn_b1'][l]), params['ffn_w2'][l],
                  r2(params['ffn_b2'][l]), r2(params['ln2_g'][l]),
                  r2(params['ln2_b'][l]))

    eps = jax.random.normal(jax.random.key(42), (N, HD), dtype=jnp.float32)
    mean, logvar, z, kl_sum = _latent(
        x, params['w_mu'], r2(params['b_mu']),
        params['w_lv'], r2(params['b_lv']), eps)
    kl = -0.5 * kl_sum[0, 0] / (N * HD)

    for s in range(S):
        nbt = jax.ops.segment_sum(z[src], dst, num_segments=N)
        nbp = jnp.stack([nbt, jnp.zeros_like(nbt)])
        z = _diff_step(z, nbp, denp0,
                       params['w_diff'][s], r2(params['b_diff'][s]))

    emb = _final(z, r2(params['fin_g']), r2(params['fin_b']),
                 params['out_w'], r2(params['out_b']))
    return emb, mean, logvar, kl, te
